# Initial kernel scaffold; baseline (speedup 1.0000x reference)
import jax
import jax.numpy as jnp
from jax.experimental import pallas as pl


def kernel(points, qff_vector, freqs):
    raise NotImplementedError("write your pallas kernel here")


# speedup vs baseline: 13.2073x; 13.2073x over previous
import jax
import jax.numpy as jnp
from jax.experimental import pallas as pl

NF = 6
C = 4
Q = 80
R = 8
XD = 3
NCOMB = NF * 2


def _qff_body(pts_ref, freqs_ref, table_ref, out_ref):
    pts = pts_ref[...]
    b = pts.shape[0]
    iota = jax.lax.broadcasted_iota(jnp.int32, (1, Q), 1).astype(jnp.float32)
    freqs = freqs_ref[...]
    for c in range(NCOMB):
        f = freqs[0, c // 2]
        acc = None
        for ax in range(XD):
            fp = pts[:, ax:ax + 1] * f
            enc = jnp.sin(fp) if (c % 2 == 0) else jnp.cos(fp)
            pos = (enc + 1.0) * (0.5 * (Q - 1))
            w = jnp.maximum(0.0, 1.0 - jnp.abs(pos - iota))
            j = c * XD + ax
            r = jnp.dot(w, table_ref[j], preferred_element_type=jnp.float32)
            acc = r if acc is None else acc * r
        red = jnp.sum(acc.reshape(b, C, R), axis=2)
        out_ref[:, c * C:(c + 1) * C] = red


def kernel(points, qff_vector, freqs):
    n = points.shape[0]
    B = 1000
    assert n % B == 0
    table = qff_vector.reshape(NCOMB * XD, C * R, Q).transpose(0, 2, 1)
    freqs2 = freqs.reshape(1, NF)
    grid = (n // B,)
    out = pl.pallas_call(
        _qff_body,
        grid=grid,
        in_specs=[
            pl.BlockSpec((B, XD), lambda i: (i, 0)),
            pl.BlockSpec((1, NF), lambda i: (0, 0)),
            pl.BlockSpec((NCOMB * XD, Q, C * R), lambda i: (0, 0, 0)),
        ],
        out_specs=pl.BlockSpec((B, NCOMB * C), lambda i: (i, 0)),
        out_shape=jax.ShapeDtypeStruct((n, NCOMB * C), jnp.float32),
    )(points, freqs2, table)
    return out


# speedup vs baseline: 17.6564x; 1.3369x over previous
import functools

import jax
import jax.numpy as jnp
from jax import lax
from jax.experimental import pallas as pl
from jax.experimental.pallas import tpu as pltpu
from jax.experimental.pallas import tpu_sc as plsc

NF = 6
C = 4
Q = 80
R = 8
XD = 3
NCOMB = NF * 2
NJ = NCOMB * XD
CR = C * R
OUTC = NCOMB * C
L = 16
NTILES = 32
PTS_PER_TILE = 3200
NPAD = NTILES * PTS_PER_TILE
CHUNK_PTS = 128
NCHUNK = PTS_PER_TILE // CHUNK_PTS
NGROUP = CHUNK_PTS // L


def _pos_body(pts_ref, freqs_ref, out_ref):
    for c in range(NCOMB):
        f = freqs_ref[0, c // 2]
        for ax in range(XD):
            x = pts_ref[ax:ax + 1, :] * f
            enc = jnp.sin(x) if (c % 2 == 0) else jnp.cos(x)
            j = c * XD + ax
            out_ref[j:j + 1, :] = (enc + 1.0) * (0.5 * (Q - 1))


def _sc_body(tab_hbm, pos_hbm, out_hbm, tab_v, pos_v, out_v):
    nc = 2
    wid = lax.axis_index("s") * nc + lax.axis_index("c")
    pltpu.sync_copy(tab_hbm, tab_v)

    def chunk_body(ck, _):
        start = pl.multiple_of(wid * PTS_PER_TILE + ck * CHUNK_PTS, 128)
        pltpu.sync_copy(pos_hbm.at[:, pl.ds(start, CHUNK_PTS)], pos_v)

        def group_body(g, _):
            o = g * L

            def comb_body(c, _):
                ws = []
                bases = []
                for ax in range(XD):
                    j = c * XD + ax
                    pv = pos_v[j, pl.ds(o, L)]
                    i0 = jnp.minimum(pv.astype(jnp.int32), Q - 2)
                    ws.append(pv - i0.astype(jnp.float32))
                    bases.append(i0 * CR + j * (Q * CR))
                accs = [jnp.zeros((L,), jnp.float32) for _ in range(C)]
                for ch in range(CR):
                    p = None
                    for ax in range(XD):
                        v0 = plsc.load_gather(tab_v, [bases[ax] + ch])
                        v1 = plsc.load_gather(tab_v, [bases[ax] + (ch + CR)])
                        lerp = v0 + ws[ax] * (v1 - v0)
                        p = lerp if p is None else p * lerp
                    accs[ch // R] = accs[ch // R] + p
                for cf in range(C):
                    out_v[c * C + cf, pl.ds(o, L)] = accs[cf]
                return 0

            return lax.fori_loop(0, NCOMB, comb_body, 0)

        lax.fori_loop(0, NGROUP, group_body, 0)
        oco = pl.multiple_of(ck * CHUNK_PTS, 128)
        pltpu.sync_copy(out_v, out_hbm.at[wid, :, pl.ds(oco, CHUNK_PTS)])
        return 0

    lax.fori_loop(0, NCHUNK, chunk_body, 0)


def kernel(points, qff_vector, freqs):
    n = points.shape[0]
    pts_t = jnp.pad(points, ((0, NPAD - n), (0, 0))).T
    freqs2 = freqs.reshape(1, NF)
    B = 2048
    pos = pl.pallas_call(
        _pos_body,
        grid=(NPAD // B,),
        in_specs=[
            pl.BlockSpec((XD, B), lambda i: (0, i)),
            pl.BlockSpec((1, NF), lambda i: (0, 0)),
        ],
        out_specs=pl.BlockSpec((NJ, B), lambda i: (0, i)),
        out_shape=jax.ShapeDtypeStruct((NJ, NPAD), jnp.float32),
    )(pts_t, freqs2)

    table = qff_vector.reshape(NJ, CR, Q).transpose(0, 2, 1).reshape(-1)
    sc = functools.partial(
        pl.kernel,
        out_type=jax.ShapeDtypeStruct((NTILES, OUTC, PTS_PER_TILE), jnp.float32),
        mesh=plsc.VectorSubcoreMesh(core_axis_name="c", subcore_axis_name="s"),
        compiler_params=pltpu.CompilerParams(needs_layout_passes=False),
        scratch_types=[
            pltpu.VMEM((NJ * Q * CR,), jnp.float32),
            pltpu.VMEM((NJ, CHUNK_PTS), jnp.float32),
            pltpu.VMEM((OUTC, CHUNK_PTS), jnp.float32),
        ],
    )(_sc_body)
    out_t = sc(table, pos)
    out = out_t.transpose(0, 2, 1).reshape(NPAD, OUTC)[:n]
    return out
